# baseline (speedup 1.0000x reference)
import jax
import jax.numpy as jnp
from jax.experimental import pallas as pl
from jax.experimental.pallas import tpu as pltpu

B, L, D, H = 128, 64, 256, 128
NUM_CLASSES, HASH_BITS = 100, 48
T = 8

_DNT = (((1,), (1,)), ((), ()))


def _dott(a, b):
    return jax.lax.dot_general(a, b, _DNT, preferred_element_type=jnp.float32)


def _dranet_kernel(seq_ref, sl_col_ref, sl_row_ref,
                   Wih_ref, Whh_ref, b_ih_ref, b_hh_ref,
                   Wq_ref, Wk_ref, Wv_ref, Wp_ref, bp_ref, Wh_ref, bh_ref,
                   pred_ref, hash_ref, att_ref,
                   h_ref, k_ref, v_ref):
    tb = pl.program_id(0)

    @pl.when(tb == 0)
    def _init():
        h_ref[...] = jnp.zeros((B, H), jnp.float32)

    xs = seq_ref[...]
    sl_col = sl_col_ref[...]
    Whh = Whh_ref[...]
    b_ih = b_ih_ref[...]
    b_hh = b_hh_ref[...]
    bcomb = b_ih + jnp.where(
        jax.lax.broadcasted_iota(jnp.int32, (1, 3 * H), 1) < 2 * H, b_hh, 0.0)
    b_hh_n = b_hh[:, 2 * H:]
    Wih = Wih_ref[...]
    Wk = Wk_ref[...]
    Wv = Wv_ref[...]

    hn = h_ref[...]
    for u in range(T):
        t = tb * T + u
        x = xs[:, u * D:(u + 1) * D]
        gi = _dott(x, Wih) + bcomb
        v_ref[pl.ds(t, 1)] = jnp.maximum(_dott(x, Wv), 0.0)[None]
        k_ref[pl.ds(t, 1)] = _dott(x, Wk)[None]

        h = hn
        gh = _dott(h, Whh)
        rz = jax.nn.sigmoid(gi[:, :2 * H] + gh[:, :2 * H])
        r = rz[:, :H]
        z = rz[:, H:]
        n = jnp.tanh(gi[:, 2 * H:] + r * (gh[:, 2 * H:] + b_hh_n))
        h_new = (1.0 - z) * n + z * h
        hn = jnp.where(t < sl_col, h_new, h)
    h_ref[...] = hn

    @pl.when(tb == L // T - 1)
    def _tail():
        query = _dott(hn, Wq_ref[...])
        dist = jnp.sum(k_ref[...] * query[None, :, :], axis=2)
        s = dist * (1.0 / jnp.sqrt(jnp.float32(H)))
        m = jnp.max(s, axis=0, keepdims=True)
        e = jnp.exp(s - m)
        pos_l = jax.lax.broadcasted_iota(jnp.int32, (L, B), 0)
        sl_row = sl_row_ref[...]
        e = jnp.where(pos_l < sl_row, e, 0.0)
        att = e / jnp.sum(e, axis=0, keepdims=True)

        out = jnp.sum(att[:, :, None] * v_ref[...], axis=0) + query

        iota_j = jax.lax.broadcasted_iota(jnp.int32, (B, B), 0)
        iota_i = jax.lax.broadcasted_iota(jnp.int32, (B, B), 1)
        before = (sl_col > sl_row) | ((sl_col == sl_row) & (iota_j < iota_i))
        rank_row = jnp.sum(before.astype(jnp.int32), axis=0, keepdims=True)
        perm = (iota_j == rank_row).astype(jnp.float32)
        att_ref[...] = jnp.dot(perm, att.T, preferred_element_type=jnp.float32)

        pred_ref[...] = _dott(out, Wp_ref[...]) + bp_ref[...]
        hash_ref[...] = jnp.tanh(_dott(out, Wh_ref[...]) + bh_ref[...])


@jax.jit
def kernel(sequence, sq_len, W_ih, W_hh, b_ih, b_hh, Wq, Wk, Wv, Wp, bp, Wh, bh):
    def c2(shape):
        return pl.BlockSpec(shape, lambda t: (0, 0))

    predict, hash_code, att_sq = pl.pallas_call(
        _dranet_kernel,
        grid=(L // T,),
        in_specs=[
            pl.BlockSpec((B, T * D), lambda t: (0, t)),
            c2((B, 1)), c2((1, B)),
            c2((3 * H, D)), c2((3 * H, H)), c2((1, 3 * H)), c2((1, 3 * H)),
            c2((H, H)), c2((H, D)), c2((H, D)),
            c2((NUM_CLASSES, H)), c2((1, NUM_CLASSES)),
            c2((HASH_BITS, H)), c2((1, HASH_BITS)),
        ],
        out_specs=[
            c2((B, NUM_CLASSES)),
            c2((B, HASH_BITS)),
            c2((B, L)),
        ],
        out_shape=[
            jax.ShapeDtypeStruct((B, NUM_CLASSES), jnp.float32),
            jax.ShapeDtypeStruct((B, HASH_BITS), jnp.float32),
            jax.ShapeDtypeStruct((B, L), jnp.float32),
        ],
        scratch_shapes=[
            pltpu.VMEM((B, H), jnp.float32),
            pltpu.VMEM((L, B, H), jnp.float32),
            pltpu.VMEM((L, B, H), jnp.float32),
        ],
        compiler_params=pltpu.CompilerParams(
            vmem_limit_bytes=100 * 1024 * 1024,
        ),
    )(sequence.reshape(B, L * D),
      sq_len.reshape(B, 1),
      sq_len.reshape(1, B),
      W_ih, W_hh,
      b_ih.reshape(1, -1), b_hh.reshape(1, -1),
      Wq, Wk, Wv,
      Wp, bp.reshape(1, -1),
      Wh, bh.reshape(1, -1))
    return predict, hash_code, att_sq
